# baseline (speedup 1.0000x reference)
import jax
import jax.numpy as jnp
from jax import lax
from jax.experimental import pallas as pl
from jax.experimental.pallas import tpu as pltpu
from jax.experimental.pallas import tpu_sc as plsc

N = 10000
E = 320000
IN_C = 128
HID_C = 256
OUT_C = 128

NC = 2
NS = 16
NW = NC * NS
EPW = E // NW
CHK = 40
K = 250
G = 25
NG = K // G
NBUF = 8
NPAD = 10112
RPT = NPAD // NS
NPAD_D = 10240
RPT_D = NPAD_D // NS

_mesh = plsc.VectorSubcoreMesh(
    core_axis_name="c", subcore_axis_name="s", num_cores=NC, num_subcores=NS
)


def _seg_body(data, src3, dst3, znd,
              parts,
              sgrp, dgrp, *rest):
    c = lax.axis_index("c")
    s = lax.axis_index("s")
    wid = s * NC + c
    rb = s * RPT
    bufs = rest[:NBUF]
    acc = rest[NBUF]
    gsems = rest[NBUF + 1:]

    pltpu.sync_copy(znd.at[pl.ds(rb, RPT)], acc.at[pl.ds(rb, RPT)])
    plsc.subcore_barrier()

    def group(g, carry):
        pltpu.sync_copy(src3.at[wid, g], sgrp)
        pltpu.sync_copy(dst3.at[wid, g], dgrp)
        pend_g = {
            b: pltpu.async_copy(data.at[sgrp.at[b]], bufs[b], gsems[b])
            for b in range(NBUF)
        }
        for u in range(G):
            b = u % NBUF
            pend_g[u].wait()
            pltpu.sync_copy(bufs[b], acc.at[dgrp.at[u]], add=True)
            if u + NBUF < G:
                pend_g[u + NBUF] = pltpu.async_copy(
                    data.at[sgrp.at[u + NBUF]], bufs[b], gsems[b])
        return carry

    lax.fori_loop(0, NG, group, 0)
    plsc.subcore_barrier()

    pltpu.sync_copy(acc.at[pl.ds(rb, RPT)], parts.at[c, pl.ds(rb, RPT)])


_seg_sum = pl.kernel(
    _seg_body,
    out_type=jax.ShapeDtypeStruct((NC, NPAD, IN_C), jnp.float32),
    mesh=_mesh,
    scratch_types=(
        [pltpu.VMEM((G, CHK), jnp.int32), pltpu.VMEM((G, CHK), jnp.int32)]
        + [pltpu.VMEM((CHK, IN_C), jnp.float32)] * NBUF
        + [pltpu.VMEM_SHARED((NPAD, IN_C), jnp.float32)]
        + [pltpu.SemaphoreType.DMA] * NBUF
    ),
)


def _deg_body(dst2, zn,
              degp,
              didx, hist, hsum, degv, dstage):
    c = lax.axis_index("c")
    s = lax.axis_index("s")
    wid = s * NC + c
    rb = s * RPT_D

    pltpu.sync_copy(zn, hist)
    pltpu.sync_copy(dst2.at[wid], didx)
    onesv = jnp.ones((16,), jnp.float32)

    def it(i, carry):
        idxv = didx[pl.ds(i * 16, 16)]
        plsc.addupdate_scatter(hist, [idxv], onesv)
        return carry

    lax.fori_loop(0, EPW // 16, it, 0)
    pltpu.sync_copy(hist, dstage.at[s])
    plsc.subcore_barrier()
    pltpu.sync_copy(dstage.at[:, pl.ds(rb, RPT_D)], hsum)

    def red(k, carry):
        v = hsum[0, pl.ds(16 * k, 16)]
        for t in range(1, NS):
            v = v + hsum[t, pl.ds(16 * k, 16)]
        degv[pl.ds(16 * k, 16)] = v
        return carry

    lax.fori_loop(0, RPT_D // 16, red, 0)
    pltpu.sync_copy(degv, degp.at[c, pl.ds(rb, RPT_D)])


_deg_sum = pl.kernel(
    _deg_body,
    out_type=jax.ShapeDtypeStruct((NC, NPAD_D), jnp.float32),
    mesh=_mesh,
    scratch_types=[
        pltpu.VMEM((EPW,), jnp.int32),
        pltpu.VMEM((NPAD_D,), jnp.float32),
        pltpu.VMEM((NS, RPT_D), jnp.float32),
        pltpu.VMEM((RPT_D,), jnp.float32),
        pltpu.VMEM_SHARED((NS, NPAD_D), jnp.float32),
    ],
    compiler_params=pltpu.CompilerParams(needs_layout_passes=False),
)

BL = 1000


def _tcr_body(x_ref, w1r_ref, b1_ref, xr_ref):
    xr_ref[...] = (jnp.dot(x_ref[...], w1r_ref[...],
                           preferred_element_type=jnp.float32) + b1_ref[...])


_tcr = pl.pallas_call(
    _tcr_body,
    grid=(N // BL,),
    in_specs=[
        pl.BlockSpec((BL, IN_C), lambda i: (i, 0)),
        pl.BlockSpec((IN_C, HID_C), lambda i: (0, 0)),
        pl.BlockSpec((1, HID_C), lambda i: (0, 0)),
    ],
    out_specs=pl.BlockSpec((BL, HID_C), lambda i: (i, 0)),
    out_shape=jax.ShapeDtypeStruct((N, HID_C), jnp.float32),
)


def _tc1_body(xr_ref, sp_ref, dp_ref, w1l_ref, w2l_ref, w2r_ref,
              p_ref, r_ref):
    deg = dp_ref[0] + dp_ref[1]
    rdeg = 1.0 / jnp.maximum(deg, 1.0)
    mean = (sp_ref[0] + sp_ref[1]) * rdeg
    h = (jnp.dot(mean, w1l_ref[...], preferred_element_type=jnp.float32)
         + xr_ref[...])
    h = jnp.maximum(h, 0.0)
    p_ref[...] = jnp.dot(h, w2l_ref[...], preferred_element_type=jnp.float32)
    r_ref[...] = jnp.dot(h, w2r_ref[...], preferred_element_type=jnp.float32)


_tc1 = pl.pallas_call(
    _tc1_body,
    grid=(N // BL,),
    in_specs=[
        pl.BlockSpec((BL, HID_C), lambda i: (i, 0)),
        pl.BlockSpec((NC, BL, IN_C), lambda i: (0, i, 0)),
        pl.BlockSpec((NC, BL, 1), lambda i: (0, i, 0)),
        pl.BlockSpec((IN_C, HID_C), lambda i: (0, 0)),
        pl.BlockSpec((HID_C, OUT_C), lambda i: (0, 0)),
        pl.BlockSpec((HID_C, OUT_C), lambda i: (0, 0)),
    ],
    out_specs=(
        pl.BlockSpec((BL, OUT_C), lambda i: (i, 0)),
        pl.BlockSpec((BL, OUT_C), lambda i: (i, 0)),
    ),
    out_shape=(
        jax.ShapeDtypeStruct((N, OUT_C), jnp.float32),
        jax.ShapeDtypeStruct((N, OUT_C), jnp.float32),
    ),
)


def _tc2_body(sp_ref, dp_ref, r_ref, b2_ref, o_ref):
    deg = dp_ref[0] + dp_ref[1]
    rdeg = 1.0 / jnp.maximum(deg, 1.0)
    o_ref[...] = (sp_ref[0] + sp_ref[1]) * rdeg + r_ref[...] + b2_ref[...]


_tc2 = pl.pallas_call(
    _tc2_body,
    grid=(N // BL,),
    in_specs=[
        pl.BlockSpec((NC, BL, OUT_C), lambda i: (0, i, 0)),
        pl.BlockSpec((NC, BL, 1), lambda i: (0, i, 0)),
        pl.BlockSpec((BL, OUT_C), lambda i: (i, 0)),
        pl.BlockSpec((1, OUT_C), lambda i: (0, 0)),
    ],
    out_specs=pl.BlockSpec((BL, OUT_C), lambda i: (i, 0)),
    out_shape=jax.ShapeDtypeStruct((N, OUT_C), jnp.float32),
)


def kernel(x, edge_index, W1_l, b1, W1_r, W2_l, b2, W2_r):
    src3 = edge_index[0].reshape(NW, NG, G, CHK)
    dst3 = edge_index[1].reshape(NW, NG, G, CHK)
    znd = jnp.zeros((NPAD, IN_C), jnp.float32)
    zn = jnp.zeros((NPAD_D,), jnp.float32)

    xr = _tcr(x, W1_r, b1.reshape(1, HID_C))
    degp = _deg_sum(edge_index[1].reshape(NW, EPW), zn).reshape(NC, NPAD_D, 1)
    s1p = _seg_sum(x, src3, dst3, znd)
    p, r = _tc1(xr, s1p, degp, W1_l, W2_l, W2_r)
    s2p = _seg_sum(p, src3, dst3, znd)
    return _tc2(s2p, degp, r, b2.reshape(1, OUT_C))


# speedup vs baseline: 1.1159x; 1.1159x over previous
import jax
import jax.numpy as jnp
from jax import lax
from jax.experimental import pallas as pl
from jax.experimental.pallas import tpu as pltpu
from jax.experimental.pallas import tpu_sc as plsc

N = 10000
E = 320000
IN_C = 128
HID_C = 256
OUT_C = 128

NC = 2
NS = 16
NW = NC * NS
EPW = E // NW
CHK = 80
K = 125
G = 25
NG = K // G
NPAD = 10112
RPT = NPAD // NS
NPAD_D = 10240
RPT_D = NPAD_D // NS

_mesh = plsc.VectorSubcoreMesh(
    core_axis_name="c", subcore_axis_name="s", num_cores=NC, num_subcores=NS
)


def _seg_body(data, src3, dst3, znd,
              parts,
              sgrp, dgrp, r0, r1, r2, r3, acc,
              sg0, sg1, sg2, sg3, ss0, ss1, ss2, ss3):
    c = lax.axis_index("c")
    s = lax.axis_index("s")
    wid = s * NC + c
    rb = s * RPT
    bufs = (r0, r1, r2, r3)
    gsems = (sg0, sg1, sg2, sg3)
    ssems = (ss0, ss1, ss2, ss3)

    pltpu.sync_copy(znd.at[pl.ds(rb, RPT)], acc.at[pl.ds(rb, RPT)])
    plsc.subcore_barrier()

    def group(g, carry):
        pltpu.sync_copy(src3.at[wid, g], sgrp)
        pltpu.sync_copy(dst3.at[wid, g], dgrp)
        pend_g = {
            b: pltpu.async_copy(data.at[sgrp.at[b]], bufs[b], gsems[b])
            for b in range(4)
        }
        for u in range(G):
            b = u % 4
            pend_g[u].wait()
            pltpu.sync_copy(bufs[b], acc.at[dgrp.at[u]], add=True)
            if u + 4 < G:
                pend_g[u + 4] = pltpu.async_copy(
                    data.at[sgrp.at[u + 4]], bufs[b], gsems[b])
        return carry

    lax.fori_loop(0, NG, group, 0)
    plsc.subcore_barrier()

    pltpu.sync_copy(acc.at[pl.ds(rb, RPT)], parts.at[c, pl.ds(rb, RPT)])


_seg_sum = pl.kernel(
    _seg_body,
    out_type=jax.ShapeDtypeStruct((NC, NPAD, IN_C), jnp.float32),
    mesh=_mesh,
    scratch_types=[
        pltpu.VMEM((G, CHK), jnp.int32),
        pltpu.VMEM((G, CHK), jnp.int32),
        pltpu.VMEM((CHK, IN_C), jnp.float32),
        pltpu.VMEM((CHK, IN_C), jnp.float32),
        pltpu.VMEM((CHK, IN_C), jnp.float32),
        pltpu.VMEM((CHK, IN_C), jnp.float32),
        pltpu.VMEM_SHARED((NPAD, IN_C), jnp.float32),
        pltpu.SemaphoreType.DMA,
        pltpu.SemaphoreType.DMA,
        pltpu.SemaphoreType.DMA,
        pltpu.SemaphoreType.DMA,
        pltpu.SemaphoreType.DMA,
        pltpu.SemaphoreType.DMA,
        pltpu.SemaphoreType.DMA,
        pltpu.SemaphoreType.DMA,
    ],
)


def _deg_body(dst2, zn,
              degp,
              didx, hist, hsum, degv, dstage):
    c = lax.axis_index("c")
    s = lax.axis_index("s")
    wid = s * NC + c
    rb = s * RPT_D

    pltpu.sync_copy(zn, hist)
    pltpu.sync_copy(dst2.at[wid], didx)
    onesv = jnp.ones((16,), jnp.float32)

    def it(i, carry):
        idxv = didx[pl.ds(i * 16, 16)]
        plsc.addupdate_scatter(hist, [idxv], onesv)
        return carry

    lax.fori_loop(0, EPW // 16, it, 0)
    pltpu.sync_copy(hist, dstage.at[s])
    plsc.subcore_barrier()
    pltpu.sync_copy(dstage.at[:, pl.ds(rb, RPT_D)], hsum)

    def red(k, carry):
        v = hsum[0, pl.ds(16 * k, 16)]
        for t in range(1, NS):
            v = v + hsum[t, pl.ds(16 * k, 16)]
        degv[pl.ds(16 * k, 16)] = v
        return carry

    lax.fori_loop(0, RPT_D // 16, red, 0)
    pltpu.sync_copy(degv, degp.at[c, pl.ds(rb, RPT_D)])


_deg_sum = pl.kernel(
    _deg_body,
    out_type=jax.ShapeDtypeStruct((NC, NPAD_D), jnp.float32),
    mesh=_mesh,
    scratch_types=[
        pltpu.VMEM((EPW,), jnp.int32),
        pltpu.VMEM((NPAD_D,), jnp.float32),
        pltpu.VMEM((NS, RPT_D), jnp.float32),
        pltpu.VMEM((RPT_D,), jnp.float32),
        pltpu.VMEM_SHARED((NS, NPAD_D), jnp.float32),
    ],
    compiler_params=pltpu.CompilerParams(needs_layout_passes=False),
)

BL = 1000


def _tcr_body(x_ref, w1r_ref, b1_ref, xr_ref):
    xr_ref[...] = (jnp.dot(x_ref[...], w1r_ref[...],
                           preferred_element_type=jnp.float32) + b1_ref[...])


_tcr = pl.pallas_call(
    _tcr_body,
    grid=(N // BL,),
    in_specs=[
        pl.BlockSpec((BL, IN_C), lambda i: (i, 0)),
        pl.BlockSpec((IN_C, HID_C), lambda i: (0, 0)),
        pl.BlockSpec((1, HID_C), lambda i: (0, 0)),
    ],
    out_specs=pl.BlockSpec((BL, HID_C), lambda i: (i, 0)),
    out_shape=jax.ShapeDtypeStruct((N, HID_C), jnp.float32),
)


def _tc1_body(xr_ref, sp_ref, dp_ref, w1l_ref, w2l_ref, w2r_ref,
              p_ref, r_ref):
    deg = dp_ref[0] + dp_ref[1]
    rdeg = 1.0 / jnp.maximum(deg, 1.0)
    mean = (sp_ref[0] + sp_ref[1]) * rdeg
    h = (jnp.dot(mean, w1l_ref[...], preferred_element_type=jnp.float32)
         + xr_ref[...])
    h = jnp.maximum(h, 0.0)
    p_ref[...] = jnp.dot(h, w2l_ref[...], preferred_element_type=jnp.float32)
    r_ref[...] = jnp.dot(h, w2r_ref[...], preferred_element_type=jnp.float32)


_tc1 = pl.pallas_call(
    _tc1_body,
    grid=(N // BL,),
    in_specs=[
        pl.BlockSpec((BL, HID_C), lambda i: (i, 0)),
        pl.BlockSpec((NC, BL, IN_C), lambda i: (0, i, 0)),
        pl.BlockSpec((NC, BL, 1), lambda i: (0, i, 0)),
        pl.BlockSpec((IN_C, HID_C), lambda i: (0, 0)),
        pl.BlockSpec((HID_C, OUT_C), lambda i: (0, 0)),
        pl.BlockSpec((HID_C, OUT_C), lambda i: (0, 0)),
    ],
    out_specs=(
        pl.BlockSpec((BL, OUT_C), lambda i: (i, 0)),
        pl.BlockSpec((BL, OUT_C), lambda i: (i, 0)),
    ),
    out_shape=(
        jax.ShapeDtypeStruct((N, OUT_C), jnp.float32),
        jax.ShapeDtypeStruct((N, OUT_C), jnp.float32),
    ),
)


def _tc2_body(sp_ref, dp_ref, r_ref, b2_ref, o_ref):
    deg = dp_ref[0] + dp_ref[1]
    rdeg = 1.0 / jnp.maximum(deg, 1.0)
    o_ref[...] = (sp_ref[0] + sp_ref[1]) * rdeg + r_ref[...] + b2_ref[...]


_tc2 = pl.pallas_call(
    _tc2_body,
    grid=(N // BL,),
    in_specs=[
        pl.BlockSpec((NC, BL, OUT_C), lambda i: (0, i, 0)),
        pl.BlockSpec((NC, BL, 1), lambda i: (0, i, 0)),
        pl.BlockSpec((BL, OUT_C), lambda i: (i, 0)),
        pl.BlockSpec((1, OUT_C), lambda i: (0, 0)),
    ],
    out_specs=pl.BlockSpec((BL, OUT_C), lambda i: (i, 0)),
    out_shape=jax.ShapeDtypeStruct((N, OUT_C), jnp.float32),
)


def kernel(x, edge_index, W1_l, b1, W1_r, W2_l, b2, W2_r):
    src3 = edge_index[0].reshape(NW, NG, G, CHK)
    dst3 = edge_index[1].reshape(NW, NG, G, CHK)
    znd = jnp.zeros((NPAD, IN_C), jnp.float32)
    zn = jnp.zeros((NPAD_D,), jnp.float32)

    xr = _tcr(x, W1_r, b1.reshape(1, HID_C))
    degp = _deg_sum(edge_index[1].reshape(NW, EPW), zn).reshape(NC, NPAD_D, 1)
    s1p = _seg_sum(x, src3, dst3, znd)
    p, r = _tc1(xr, s1p, degp, W1_l, W2_l, W2_r)
    s2p = _seg_sum(p, src3, dst3, znd)
    return _tc2(s2p, degp, r, b2.reshape(1, OUT_C))
